# baseline (speedup 1.0000x reference)
import dataclasses
import functools

import jax
import jax.numpy as jnp
from jax.experimental import pallas as pl
from jax.experimental.pallas import tpu as pltpu
from jax.experimental.pallas import tpu_sc as plsc

_ROWS = 16384
_C = 2048
_L = 16
_RT = 4


def kernel(z, perm):
    perm32 = perm.astype(jnp.int32)
    mesh = plsc.VectorSubcoreMesh(
        core_axis_name="core", subcore_axis_name="subcore"
    )

    cp = pltpu.CompilerParams()
    if "needs_layout_passes" in pltpu.CompilerParams.__dataclass_fields__:
        cp = dataclasses.replace(cp, needs_layout_passes=False)

    @functools.partial(
        pl.kernel,
        out_type=jax.ShapeDtypeStruct((_ROWS, _C), jnp.float32),
        mesh=mesh,
        compiler_params=cp,
        scratch_types=[
            pltpu.VMEM((_C,), jnp.int32),
            pltpu.SemaphoreType.DMA,
        ],
    )
    def run(z_hbm, perm_hbm, out_hbm, perm_v, sem):
        pltpu.async_copy(perm_hbm, perm_v, sem).wait()

        def tile_body(z_vmem, o_vmem):
            @plsc.parallel_loop(0, _C // _L, unroll=8)
            def _(cb):
                col = perm_v[pl.ds(cb * _L, _L)]
                for r in range(_RT):
                    rowidx = jnp.full((_L,), r, jnp.int32)
                    o_vmem[r, pl.ds(cb * _L, _L)] = plsc.load_gather(
                        z_vmem, [rowidx, col]
                    )

        pltpu.emit_pipeline(
            tile_body,
            grid=(_ROWS // _RT,),
            in_specs=[pl.BlockSpec((_RT, _C), lambda i: (i, 0))],
            out_specs=[pl.BlockSpec((_RT, _C), lambda i: (i, 0))],
            core_axis_name=("core", "subcore"),
            dimension_semantics=(pltpu.PARALLEL,),
        )(z_hbm, out_hbm)

    z_out = run(z, perm32)
    return (z_out, jnp.zeros((), z.dtype))


# speedup vs baseline: 1.1250x; 1.1250x over previous
import dataclasses
import functools

import jax
import jax.numpy as jnp
from jax.experimental import pallas as pl
from jax.experimental.pallas import tpu as pltpu
from jax.experimental.pallas import tpu_sc as plsc

_ROWS = 16384
_C = 2048
_L = 16
_RT = 8


def kernel(z, perm):
    perm32 = perm.astype(jnp.int32)
    mesh = plsc.VectorSubcoreMesh(
        core_axis_name="core", subcore_axis_name="subcore"
    )

    cp = pltpu.CompilerParams()
    if "needs_layout_passes" in pltpu.CompilerParams.__dataclass_fields__:
        cp = dataclasses.replace(cp, needs_layout_passes=False)

    @functools.partial(
        pl.kernel,
        out_type=jax.ShapeDtypeStruct((_ROWS, _C), jnp.float32),
        mesh=mesh,
        compiler_params=cp,
        scratch_types=[
            pltpu.VMEM((_C,), jnp.int32),
            pltpu.SemaphoreType.DMA,
        ],
    )
    def run(z_hbm, perm_hbm, out_hbm, perm_v, sem):
        pltpu.async_copy(perm_hbm, perm_v, sem).wait()

        def tile_body(z_vmem, o_vmem):
            @plsc.parallel_loop(0, _C // _L, unroll=16)
            def _(cb):
                col = perm_v[pl.ds(cb * _L, _L)]
                for r in range(_RT):
                    rowidx = jnp.full((_L,), r, jnp.int32)
                    o_vmem[r, pl.ds(cb * _L, _L)] = plsc.load_gather(
                        z_vmem, [rowidx, col]
                    )

        pltpu.emit_pipeline(
            tile_body,
            grid=(_ROWS // _RT,),
            in_specs=[pl.BlockSpec((_RT, _C), lambda i: (i, 0))],
            out_specs=[pl.BlockSpec((_RT, _C), lambda i: (i, 0))],
            core_axis_name=("core", "subcore"),
            dimension_semantics=(pltpu.PARALLEL,),
        )(z_hbm, out_hbm)

    z_out = run(z, perm32)
    return (z_out, jnp.zeros((), z.dtype))


# speedup vs baseline: 1.2133x; 1.0785x over previous
import dataclasses
import functools

import jax
import jax.numpy as jnp
from jax.experimental import pallas as pl
from jax.experimental.pallas import tpu as pltpu
from jax.experimental.pallas import tpu_sc as plsc

_ROWS = 16384
_C = 2048
_L = 16
_RT = 8


def kernel(z, perm):
    perm32 = perm.astype(jnp.int32)
    mesh = plsc.VectorSubcoreMesh(
        core_axis_name="core", subcore_axis_name="subcore"
    )

    cp = pltpu.CompilerParams()
    if "needs_layout_passes" in pltpu.CompilerParams.__dataclass_fields__:
        cp = dataclasses.replace(cp, needs_layout_passes=False)

    @functools.partial(
        pl.kernel,
        out_type=jax.ShapeDtypeStruct((_ROWS, _C), jnp.float32),
        mesh=mesh,
        compiler_params=cp,
        scratch_types=[
            pltpu.VMEM((_C,), jnp.int32),
            pltpu.SemaphoreType.DMA,
        ],
    )
    def run(z_hbm, perm_hbm, out_hbm, perm_v, sem):
        pltpu.async_copy(perm_hbm, perm_v, sem).wait()

        def tile_body(z_vmem, o_vmem):
            @plsc.parallel_loop(0, _C // _L, unroll=8)
            def _(cb):
                col = perm_v[pl.ds(cb * _L, _L)]
                for r in range(_RT):
                    rowidx = jnp.full((_L,), r, jnp.int32)
                    o_vmem[r, pl.ds(cb * _L, _L)] = plsc.load_gather(
                        z_vmem, [rowidx, col]
                    )

        pltpu.emit_pipeline(
            tile_body,
            grid=(_ROWS // _RT,),
            in_specs=[
                pl.BlockSpec(
                    (_RT, _C),
                    lambda i: (i, 0),
                    pipeline_mode=pl.Buffered(buffer_count=3),
                )
            ],
            out_specs=[pl.BlockSpec((_RT, _C), lambda i: (i, 0))],
            core_axis_name=("core", "subcore"),
            dimension_semantics=(pltpu.PARALLEL,),
        )(z_hbm, out_hbm)

    z_out = run(z, perm32)
    return (z_out, jnp.zeros((), z.dtype))


# speedup vs baseline: 1.2183x; 1.0041x over previous
import dataclasses
import functools

import jax
import jax.numpy as jnp
from jax.experimental import pallas as pl
from jax.experimental.pallas import tpu as pltpu
from jax.experimental.pallas import tpu_sc as plsc

_ROWS = 16384
_C = 2048
_L = 16
_RT = 8


def kernel(z, perm):
    perm32 = perm.astype(jnp.int32)
    mesh = plsc.VectorSubcoreMesh(
        core_axis_name="core", subcore_axis_name="subcore"
    )

    cp = pltpu.CompilerParams()
    if "needs_layout_passes" in pltpu.CompilerParams.__dataclass_fields__:
        cp = dataclasses.replace(cp, needs_layout_passes=False)

    @functools.partial(
        pl.kernel,
        out_type=jax.ShapeDtypeStruct((_ROWS, _C), jnp.float32),
        mesh=mesh,
        compiler_params=cp,
        scratch_types=[
            pltpu.VMEM((_C,), jnp.int32),
            pltpu.SemaphoreType.DMA,
        ],
    )
    def run(z_hbm, perm_hbm, out_hbm, perm_v, sem):
        pltpu.async_copy(perm_hbm, perm_v, sem).wait()

        def tile_body(z_vmem, o_vmem):
            @plsc.parallel_loop(0, _C // _L, unroll=8)
            def _(cb):
                col = perm_v[pl.ds(cb * _L, _L)]
                for r in range(_RT):
                    rowidx = jnp.full((_L,), r, jnp.int32)
                    o_vmem[r, pl.ds(cb * _L, _L)] = plsc.load_gather(
                        z_vmem, [rowidx, col]
                    )

        pltpu.emit_pipeline(
            tile_body,
            grid=(_ROWS // _RT,),
            in_specs=[
                pl.BlockSpec(
                    (_RT, _C),
                    lambda i: (i, 0),
                    pipeline_mode=pl.Buffered(buffer_count=4),
                )
            ],
            out_specs=[pl.BlockSpec((_RT, _C), lambda i: (i, 0))],
            core_axis_name=("core", "subcore"),
            dimension_semantics=(pltpu.PARALLEL,),
        )(z_hbm, out_hbm)

    z_out = run(z, perm32)
    return (z_out, jnp.zeros((), z.dtype))
